# baseline (speedup 1.0000x reference)
import functools

import jax
import jax.numpy as jnp
from jax import lax
from jax.experimental import pallas as pl
from jax.experimental.pallas import tpu as pltpu
from jax.experimental.pallas import tpu_sc as plsc

_BATCH = 16384
_FIELDS = 26
_DIM = 64
_N = _BATCH * _FIELDS
_NUM_WORKERS = 32
_PER_WORKER = _N // _NUM_WORKERS
_CHUNK = 512
_NUM_CHUNKS = _PER_WORKER // _CHUNK

_mesh = plsc.VectorSubcoreMesh(core_axis_name="c", subcore_axis_name="s")


@functools.partial(
    pl.kernel,
    out_type=jax.ShapeDtypeStruct((_N, _DIM), jnp.float32),
    mesh=_mesh,
    scratch_types=[
        pltpu.VMEM((_CHUNK,), jnp.int32),
        pltpu.VMEM((_CHUNK, _DIM), jnp.float32),
        pltpu.SemaphoreType.DMA,
    ],
    compiler_params=pltpu.CompilerParams(use_tc_tiling_on_sc=False),
)
def _gather_kernel(keys_hbm, table_hbm, out_hbm, idx_v, rows_v, sem):
    wid = lax.axis_index("s") * 2 + lax.axis_index("c")
    base = wid * _PER_WORKER

    def body(n, carry):
        off = base + n * _CHUNK
        pltpu.sync_copy(keys_hbm.at[pl.ds(off, _CHUNK)], idx_v)
        pltpu.async_copy(table_hbm.at[idx_v], rows_v, sem).wait()
        pltpu.sync_copy(rows_v, out_hbm.at[pl.ds(off, _CHUNK)])
        return carry

    lax.fori_loop(0, _NUM_CHUNKS, body, 0)


def kernel(keys, table):
    flat = keys.reshape(-1).astype(jnp.int32)
    out = _gather_kernel(flat, table)
    return out.reshape(_BATCH, _FIELDS, _DIM)


# speedup vs baseline: 1.0231x; 1.0231x over previous
import functools

import jax
import jax.numpy as jnp
from jax import lax
from jax.experimental import pallas as pl
from jax.experimental.pallas import tpu as pltpu
from jax.experimental.pallas import tpu_sc as plsc

_BATCH = 16384
_FIELDS = 26
_DIM = 64
_N = _BATCH * _FIELDS
_NUM_WORKERS = 32
_PER_WORKER = _N // _NUM_WORKERS
_CHUNK = 512
_NUM_CHUNKS = _PER_WORKER // _CHUNK
_NBUF = 2
_NOUTER = _NUM_CHUNKS // _NBUF

_mesh = plsc.VectorSubcoreMesh(core_axis_name="c", subcore_axis_name="s")


@functools.partial(
    pl.kernel,
    out_type=jax.ShapeDtypeStruct((_N, _DIM), jnp.float32),
    mesh=_mesh,
    scratch_types=[
        pltpu.VMEM((_NUM_CHUNKS, _CHUNK), jnp.int32),
        pltpu.VMEM((_NBUF, _CHUNK, _DIM), jnp.float32),
        pltpu.SemaphoreType.DMA,
        pltpu.SemaphoreType.DMA,
        pltpu.SemaphoreType.DMA,
        pltpu.SemaphoreType.DMA,
    ],
    compiler_params=pltpu.CompilerParams(use_tc_tiling_on_sc=False),
)
def _gather_kernel(keys_hbm, table_hbm, out_hbm, idx_v, rows_v, gs0, gs1, os0, os1):
    gsem = (gs0, gs1)
    osem = (os0, os1)
    wid = lax.axis_index("s") * 2 + lax.axis_index("c")
    base = wid * _PER_WORKER
    pltpu.sync_copy(keys_hbm.at[wid], idx_v)

    def gather(n, b):
        return pltpu.make_async_copy(
            table_hbm.at[idx_v.at[n]], rows_v.at[b], gsem[b])

    def store(n, b):
        return pltpu.make_async_copy(
            rows_v.at[b], out_hbm.at[pl.ds(base + n * _CHUNK, _CHUNK)], osem[b])

    for b in range(_NBUF):
        gather(b, b).start()

    def body(i, carry):
        for b in range(_NBUF):
            n = i * _NBUF + b
            gather(n, b).wait()
            store(n, b).start()
        for b in range(_NBUF):
            n = i * _NBUF + b
            store(n, b).wait()
            gather(n + _NBUF, b).start()
        return carry

    lax.fori_loop(0, _NOUTER - 1, body, 0)

    for b in range(_NBUF):
        n = (_NOUTER - 1) * _NBUF + b
        gather(n, b).wait()
        store(n, b).start()
    for b in range(_NBUF):
        n = (_NOUTER - 1) * _NBUF + b
        store(n, b).wait()


def kernel(keys, table):
    keys3 = keys.reshape(_NUM_WORKERS, _NUM_CHUNKS, _CHUNK).astype(jnp.int32)
    out = _gather_kernel(keys3, table)
    return out.reshape(_BATCH, _FIELDS, _DIM)
